# baseline (speedup 1.0000x reference)
import functools

import jax
import jax.numpy as jnp
from jax import lax
from jax.experimental import pallas as pl
from jax.experimental.pallas import tpu as pltpu
from jax.experimental.pallas import tpu_sc as plsc

MEM = 10000
BATCH = 1024
ROW = 4 * 84 * 84
NC = 2
NS = 16
NW = NC * NS
RPW = BATCH // NW
NSLOT = 4
LANES = 16
NVEC = BATCH // LANES
ALPHA_CORR = 0.1
BETA = 0.5

_f32 = jnp.float32
_i32 = jnp.int32


def _rsqrt(x):
    xi = plsc.bitcast(x, _i32)
    yi = jnp.int32(0x5F3759DF) - lax.shift_right_arithmetic(xi, 1)
    y = plsc.bitcast(yi, _f32)
    for _ in range(3):
        y = y * (1.5 - 0.5 * x * y * y)
    return y


def _body(s0, s1, act, rew, don, pri, idx2, idxf, err,
          o_s0, o_a, o_r, o_d, o_s1, o_w, o_p,
          rows, idxv, table, idx_all, vals, wbuf,
          g0, g1, g2, g3, w0, w1, w2, w3):
    gsems = (g0, g1, g2, g3)
    wsems = (w0, w1, w2, w3)
    c = lax.axis_index("c")
    s = lax.axis_index("s")
    wid = s * NC + c
    base = wid * RPW
    lane = lax.iota(_i32, LANES)

    @pl.when(wid == 0)
    def _weights_and_scatter():
        pltpu.sync_copy(pri, table)
        pltpu.sync_copy(idxf, idx_all)
        pltpu.sync_copy(err, vals)
        m = jnp.zeros((LANES,), _f32)
        for v in range(NVEC):
            iv = idx_all[pl.ds(v * LANES, LANES)]
            p = plsc.bitcast(plsc.load_gather(table, [iv]), _f32)
            w = _rsqrt(jnp.float32(MEM) * p)
            wbuf[pl.ds(v * LANES, LANES)] = plsc.bitcast(w, _i32)
            m = jnp.maximum(m, w)
        mx = jnp.max(m) * jnp.ones((LANES,), _f32)
        for v in range(NVEC):
            w = plsc.bitcast(wbuf[pl.ds(v * LANES, LANES)], _f32)
            wbuf[pl.ds(v * LANES, LANES)] = plsc.bitcast(w / mx, _i32)
        pltpu.sync_copy(wbuf, o_w)
        for v in range(NVEC):
            iv = idx_all[pl.ds(v * LANES, LANES)]
            e = plsc.bitcast(vals[pl.ds(v * LANES, LANES)], _f32)
            nv = plsc.bitcast(jnp.abs(e) + ALPHA_CORR, _i32)
            for l in range(LANES):
                plsc.store_scatter(table, [iv], nv, mask=lane == l)
        pltpu.sync_copy(table, o_p)

    def _small_gather(src_hbm, dst_hbm):
        pltpu.sync_copy(src_hbm, table)
        pltpu.sync_copy(idxf, idx_all)
        for v in range(NVEC):
            iv = idx_all[pl.ds(v * LANES, LANES)]
            wbuf[pl.ds(v * LANES, LANES)] = plsc.load_gather(table, [iv])
        pltpu.sync_copy(wbuf, dst_hbm)

    @pl.when(wid == 1)
    def _actions():
        _small_gather(act, o_a)

    @pl.when(wid == 2)
    def _rewards():
        _small_gather(rew, o_r)

    @pl.when(wid == 3)
    def _done():
        _small_gather(don, o_d)

    pltpu.sync_copy(idx2.at[pl.ds(base, RPW)], idxv)

    def pump(src, dst):
        gdescs = [None] * NSLOT
        wdescs = [None] * NSLOT
        for j in range(NSLOT):
            gdescs[j] = pltpu.async_copy(
                src.at[idxv.at[j]], rows.at[pl.ds(j, 1)], gsems[j])
        for j in range(RPW):
            sl = j % NSLOT
            gdescs[sl].wait()
            wdescs[sl] = pltpu.async_copy(
                rows.at[pl.ds(sl, 1)], dst.at[pl.ds(base + j, 1)], wsems[sl])
            nxt = j + NSLOT
            if nxt < RPW:
                wdescs[sl].wait()
                gdescs[sl] = pltpu.async_copy(
                    src.at[idxv.at[nxt]], rows.at[pl.ds(sl, 1)], gsems[sl])
        for j in range(RPW - NSLOT, RPW):
            wdescs[j % NSLOT].wait()

    pump(s0, o_s0)
    pump(s1, o_s1)


_sdt = jax.ShapeDtypeStruct

_replay = functools.partial(
    pl.kernel,
    out_type=(
        _sdt((BATCH, ROW), _f32),
        _sdt((BATCH,), _i32),
        _sdt((BATCH,), _i32),
        _sdt((BATCH,), _i32),
        _sdt((BATCH, ROW), _f32),
        _sdt((BATCH,), _i32),
        _sdt((MEM,), _i32),
    ),
    mesh=plsc.VectorSubcoreMesh(
        core_axis_name="c", subcore_axis_name="s",
        num_cores=NC, num_subcores=NS),
    scratch_types=[
        pltpu.VMEM((NSLOT, ROW), _f32),
        pltpu.VMEM((RPW, 1), _i32),
        pltpu.VMEM((MEM,), _i32),
        pltpu.VMEM((BATCH,), _i32),
        pltpu.VMEM((BATCH,), _i32),
        pltpu.VMEM((BATCH,), _i32),
    ] + [pltpu.SemaphoreType.DMA] * 8,
    compiler_params=pltpu.CompilerParams(
        needs_layout_passes=False, use_tc_tiling_on_sc=False),
)(_body)


def kernel(state0_buffer, actions_buffer, rewards_buffer, done_buffer,
           state1_buffer, priorities, indices, errors):
    s0 = state0_buffer.reshape(MEM, ROW)
    s1 = state1_buffer.reshape(MEM, ROW)
    bc_i = lambda x: lax.bitcast_convert_type(x, _i32)
    bc_f = lambda x: lax.bitcast_convert_type(x, _f32)
    idx = indices.astype(_i32)
    s0o, ao, ro, do_, s1o, wo, po = _replay(
        s0, s1, actions_buffer.astype(_i32), bc_i(rewards_buffer),
        bc_i(done_buffer), bc_i(priorities), idx.reshape(BATCH, 1), idx,
        bc_i(errors))
    return (s0o.reshape(BATCH, 4, 84, 84), ao, bc_f(ro), bc_f(do_),
            s1o.reshape(BATCH, 4, 84, 84), bc_f(wo), bc_f(po))


# speedup vs baseline: 1.8902x; 1.8902x over previous
import functools

import jax
import jax.numpy as jnp
from jax import lax
from jax.experimental import pallas as pl
from jax.experimental.pallas import tpu as pltpu
from jax.experimental.pallas import tpu_sc as plsc

MEM = 10000
BATCH = 1024
C = 4
YX = 84 * 84
BLK = 16
NBLK = YX // BLK
NC = 2
NS = 16
NW = NC * NS
MAXB = -(-NBLK // NW)
LANES = 16
NVEC = BATCH // LANES
SC_ROWS = BATCH * C
CORR = 0.1

_f32 = jnp.float32
_i32 = jnp.int32


def _rsqrt(x):
    xi = plsc.bitcast(x, _i32)
    yi = jnp.int32(0x5F3759DF) - lax.shift_right_arithmetic(xi, 1)
    y = plsc.bitcast(yi, _f32)
    for _ in range(3):
        y = y * (1.5 - 0.5 * x * y * y)
    return y


def _extract_blocks(xt, out, idx_v, pieces, accum, g0, g1, ws, tid):
    lane4 = lax.iota(_i32, LANES) * 4

    def issue(yx, half, slot, sem):
        return pltpu.async_copy(
            xt.at[yx, pl.ds(half * 2, 2), :], pieces.at[slot], sem)

    def gather_piece(slot, c0, r, col_vec):
        dc0 = jnp.zeros((LANES,), _i32)
        dc1 = jnp.ones((LANES,), _i32)

        def vbody(v, _):
            iv = idx_v[pl.ds(v * LANES, LANES)]
            base = v * 64 + c0
            g = plsc.load_gather(pieces.at[slot], [dc0, iv])
            plsc.store_scatter(accum, [lane4 + base, col_vec], g)
            g2 = plsc.load_gather(pieces.at[slot], [dc1, iv])
            plsc.store_scatter(accum, [lane4 + (base + 1), col_vec], g2)
            return 0

        lax.fori_loop(0, NVEC, vbody, 0)

    def block_body(j, _):
        b = tid + j * NW

        @pl.when(b < NBLK)
        def _():
            issue(b * BLK, 0, 0, g0)
            @pl.when(j > 0)
            def _():
                pltpu.make_async_copy(accum, out.at[:, 0, :], ws).wait()

            def rbody(r, _):
                yx = b * BLK + r
                col = jnp.full((LANES,), r, _i32)
                d1 = issue(yx, 1, 1, g1)
                pltpu.make_async_copy(xt.at[yx, pl.ds(0, 2), :],
                                      pieces.at[0], g0).wait()
                gather_piece(0, 0, r, col)

                @pl.when(r < BLK - 1)
                def _():
                    issue(yx + 1, 0, 0, g0)

                d1.wait()
                gather_piece(1, 2, r, col)
                return 0

            lax.fori_loop(0, BLK, rbody, 0)
            pltpu.async_copy(accum, out.at[:, b, :], ws)

        return 0

    lax.fori_loop(0, MAXB, block_body, 0)
    pltpu.make_async_copy(accum, out.at[:, 0, :], ws).wait()


def _body0(xt, pri, idxf, err, act, rew, don,
           out, o_a, o_r, o_d, o_w, o_p,
           idx_v, pieces, accum, table, vals, wbuf,
           g0, g1, ws):
    c = lax.axis_index("c")
    s = lax.axis_index("s")
    tid = s * NC + c
    lane = lax.iota(_i32, LANES)

    pltpu.sync_copy(idxf, idx_v)

    @pl.when(tid == 0)
    def _weights_and_scatter():
        pltpu.sync_copy(pri, table)
        pltpu.sync_copy(err, vals)
        m = jnp.zeros((LANES,), _f32)
        for v in range(NVEC):
            iv = idx_v[pl.ds(v * LANES, LANES)]
            p = plsc.bitcast(plsc.load_gather(table, [iv]), _f32)
            w = _rsqrt(jnp.float32(MEM) * p)
            wbuf[pl.ds(v * LANES, LANES)] = plsc.bitcast(w, _i32)
            m = jnp.maximum(m, w)
        mx = jnp.max(m) * jnp.ones((LANES,), _f32)
        for v in range(NVEC):
            w = plsc.bitcast(wbuf[pl.ds(v * LANES, LANES)], _f32)
            wbuf[pl.ds(v * LANES, LANES)] = plsc.bitcast(w / mx, _i32)
        pltpu.sync_copy(wbuf, o_w)
        for v in range(NVEC):
            iv = idx_v[pl.ds(v * LANES, LANES)]
            e = plsc.bitcast(vals[pl.ds(v * LANES, LANES)], _f32)
            nv = plsc.bitcast(jnp.abs(e) + CORR, _i32)
            for l in range(LANES):
                plsc.store_scatter(table, [iv], nv, mask=lane == l)
        pltpu.sync_copy(table, o_p)

    def _small_gather(src_hbm, dst_hbm):
        pltpu.sync_copy(src_hbm, table)
        for v in range(NVEC):
            iv = idx_v[pl.ds(v * LANES, LANES)]
            wbuf[pl.ds(v * LANES, LANES)] = plsc.load_gather(table, [iv])
        pltpu.sync_copy(wbuf, dst_hbm)

    @pl.when(tid == 1)
    def _actions():
        _small_gather(act, o_a)

    @pl.when(tid == 2)
    def _rewards():
        _small_gather(rew, o_r)

    @pl.when(tid == 3)
    def _done():
        _small_gather(don, o_d)

    _extract_blocks(xt, out, idx_v, pieces, accum, g0, g1, ws, tid)


def _body1(xt, idxf, out, idx_v, pieces, accum, g0, g1, ws):
    c = lax.axis_index("c")
    s = lax.axis_index("s")
    tid = s * NC + c
    pltpu.sync_copy(idxf, idx_v)
    _extract_blocks(xt, out, idx_v, pieces, accum, g0, g1, ws, tid)


_sdt = jax.ShapeDtypeStruct
_mesh = plsc.VectorSubcoreMesh(
    core_axis_name="c", subcore_axis_name="s",
    num_cores=NC, num_subcores=NS)
_params = pltpu.CompilerParams(
    needs_layout_passes=False, use_tc_tiling_on_sc=False)

_extract0 = functools.partial(
    pl.kernel,
    out_type=(
        _sdt((SC_ROWS, NBLK, BLK), _f32),
        _sdt((BATCH,), _i32),
        _sdt((BATCH,), _i32),
        _sdt((BATCH,), _i32),
        _sdt((BATCH,), _i32),
        _sdt((MEM,), _i32),
    ),
    mesh=_mesh,
    scratch_types=[
        pltpu.VMEM((BATCH,), _i32),
        pltpu.VMEM((2, 2, MEM), _f32),
        pltpu.VMEM((SC_ROWS, BLK), _f32),
        pltpu.VMEM((MEM,), _i32),
        pltpu.VMEM((BATCH,), _i32),
        pltpu.VMEM((BATCH,), _i32),
    ] + [pltpu.SemaphoreType.DMA] * 3,
    compiler_params=_params,
)(_body0)

_extract1 = functools.partial(
    pl.kernel,
    out_type=_sdt((SC_ROWS, NBLK, BLK), _f32),
    mesh=_mesh,
    scratch_types=[
        pltpu.VMEM((BATCH,), _i32),
        pltpu.VMEM((2, 2, MEM), _f32),
        pltpu.VMEM((SC_ROWS, BLK), _f32),
    ] + [pltpu.SemaphoreType.DMA] * 3,
    compiler_params=_params,
)(_body1)


def kernel(state0_buffer, actions_buffer, rewards_buffer, done_buffer,
           state1_buffer, priorities, indices, errors):
    bc_i = lambda x: lax.bitcast_convert_type(x, _i32)
    bc_f = lambda x: lax.bitcast_convert_type(x, _f32)
    xt0 = jnp.transpose(state0_buffer, (2, 3, 1, 0)).reshape(YX, C, MEM)
    xt1 = jnp.transpose(state1_buffer, (2, 3, 1, 0)).reshape(YX, C, MEM)
    idx = indices.astype(_i32)
    s0o, ao, ro, do_, wo, po = _extract0(
        xt0, bc_i(priorities), idx, bc_i(errors),
        actions_buffer.astype(_i32), bc_i(rewards_buffer),
        bc_i(done_buffer))
    s1o = _extract1(xt1, idx)
    return (s0o.reshape(BATCH, C, 84, 84), ao, bc_f(ro), bc_f(do_),
            s1o.reshape(BATCH, C, 84, 84), bc_f(wo), bc_f(po))


# speedup vs baseline: 2.0969x; 1.1093x over previous
import functools

import jax
import jax.numpy as jnp
from jax import lax
from jax.experimental import pallas as pl
from jax.experimental.pallas import tpu as pltpu
from jax.experimental.pallas import tpu_sc as plsc

MEM = 10000
BATCH = 1024
C = 4
YX = 84 * 84
BLK = 16
NBLK = YX // BLK
NC = 2
NS = 16
NW = NC * NS
MAXB = -(-NBLK // NW)
LANES = 16
NVEC = BATCH // LANES
SLABW = BATCH * C
BLKW = BLK * SLABW
OUTW = YX * SLABW
CORR = 0.1

_f32 = jnp.float32
_i32 = jnp.int32


def _rsqrt(x):
    xi = plsc.bitcast(x, _i32)
    yi = jnp.int32(0x5F3759DF) - lax.shift_right_arithmetic(xi, 1)
    y = plsc.bitcast(yi, _f32)
    for _ in range(3):
        y = y * (1.5 - 0.5 * x * y * y)
    return y


def _extract_blocks(xt, out, idx_v, pieces, accum, g0, g1, ws, tid):
    def issue(yx, half, slot, sem):
        return pltpu.async_copy(
            xt.at[yx, pl.ds(half * 2, 2), :], pieces.at[slot], sem)

    def gather_piece(slot, c0, roff):
        dc0 = jnp.zeros((LANES,), _i32)
        dc1 = jnp.ones((LANES,), _i32)

        def vbody(v, _):
            iv = idx_v[pl.ds(v * LANES, LANES)]
            base = roff + ((v >> 3) * 512 + (v & 7) * 16) + c0 * 128
            g = plsc.load_gather(pieces.at[slot], [dc0, iv])
            accum[pl.ds(base, LANES)] = g
            g2 = plsc.load_gather(pieces.at[slot], [dc1, iv])
            accum[pl.ds(base + 128, LANES)] = g2
            return 0

        lax.fori_loop(0, NVEC, vbody, 0, unroll=4)

    def block_body(j, _):
        b = tid + j * NW

        @pl.when(b < NBLK)
        def _():
            issue(b * BLK, 0, 0, g0)
            @pl.when(j > 0)
            def _():
                pltpu.make_async_copy(accum, out.at[pl.ds(0, BLKW)], ws).wait()

            def rbody(r, _):
                yx = b * BLK + r
                roff = r * 4096
                d1 = issue(yx, 1, 1, g1)
                pltpu.make_async_copy(xt.at[yx, pl.ds(0, 2), :],
                                      pieces.at[0], g0).wait()
                gather_piece(0, 0, roff)

                @pl.when(r < BLK - 1)
                def _():
                    issue(yx + 1, 0, 0, g0)

                d1.wait()
                gather_piece(1, 2, roff)
                return 0

            lax.fori_loop(0, BLK, rbody, 0)
            pltpu.async_copy(accum, out.at[pl.ds(b * BLKW, BLKW)], ws)

        return 0

    lax.fori_loop(0, MAXB, block_body, 0)
    pltpu.make_async_copy(accum, out.at[pl.ds(0, BLKW)], ws).wait()


def _body0(xt, pri, idxf, err, act, rew, don,
           out, o_a, o_r, o_d, o_w, o_p,
           idx_v, pieces, accum, table, vals, wbuf,
           g0, g1, ws):
    c = lax.axis_index("c")
    s = lax.axis_index("s")
    tid = s * NC + c
    lane = lax.iota(_i32, LANES)

    pltpu.sync_copy(idxf, idx_v)

    @pl.when(tid == 0)
    def _weights_and_scatter():
        pltpu.sync_copy(pri, table)
        pltpu.sync_copy(err, vals)
        m = jnp.zeros((LANES,), _f32)
        for v in range(NVEC):
            iv = idx_v[pl.ds(v * LANES, LANES)]
            p = plsc.bitcast(plsc.load_gather(table, [iv]), _f32)
            w = _rsqrt(jnp.float32(MEM) * p)
            wbuf[pl.ds(v * LANES, LANES)] = plsc.bitcast(w, _i32)
            m = jnp.maximum(m, w)
        mx = jnp.max(m) * jnp.ones((LANES,), _f32)
        for v in range(NVEC):
            w = plsc.bitcast(wbuf[pl.ds(v * LANES, LANES)], _f32)
            wbuf[pl.ds(v * LANES, LANES)] = plsc.bitcast(w / mx, _i32)
        pltpu.sync_copy(wbuf, o_w)
        for v in range(NVEC):
            iv = idx_v[pl.ds(v * LANES, LANES)]
            e = plsc.bitcast(vals[pl.ds(v * LANES, LANES)], _f32)
            nv = plsc.bitcast(jnp.abs(e) + CORR, _i32)
            for l in range(LANES):
                plsc.store_scatter(table, [iv], nv, mask=lane == l)
        pltpu.sync_copy(table, o_p)

    def _small_gather(src_hbm, dst_hbm):
        pltpu.sync_copy(src_hbm, table)
        for v in range(NVEC):
            iv = idx_v[pl.ds(v * LANES, LANES)]
            wbuf[pl.ds(v * LANES, LANES)] = plsc.load_gather(table, [iv])
        pltpu.sync_copy(wbuf, dst_hbm)

    @pl.when(tid == 1)
    def _actions():
        _small_gather(act, o_a)

    @pl.when(tid == 2)
    def _rewards():
        _small_gather(rew, o_r)

    @pl.when(tid == 3)
    def _done():
        _small_gather(don, o_d)

    _extract_blocks(xt, out, idx_v, pieces, accum, g0, g1, ws, tid)


def _body1(xt, idxf, out, idx_v, pieces, accum, g0, g1, ws):
    c = lax.axis_index("c")
    s = lax.axis_index("s")
    tid = s * NC + c
    pltpu.sync_copy(idxf, idx_v)
    _extract_blocks(xt, out, idx_v, pieces, accum, g0, g1, ws, tid)


_sdt = jax.ShapeDtypeStruct
_mesh = plsc.VectorSubcoreMesh(
    core_axis_name="c", subcore_axis_name="s",
    num_cores=NC, num_subcores=NS)
_params = pltpu.CompilerParams(
    needs_layout_passes=False, use_tc_tiling_on_sc=False)

_extract0 = functools.partial(
    pl.kernel,
    out_type=(
        _sdt((OUTW,), _f32),
        _sdt((BATCH,), _i32),
        _sdt((BATCH,), _i32),
        _sdt((BATCH,), _i32),
        _sdt((BATCH,), _i32),
        _sdt((MEM,), _i32),
    ),
    mesh=_mesh,
    scratch_types=[
        pltpu.VMEM((BATCH,), _i32),
        pltpu.VMEM((2, 2, MEM), _f32),
        pltpu.VMEM((BLKW,), _f32),
        pltpu.VMEM((MEM,), _i32),
        pltpu.VMEM((BATCH,), _i32),
        pltpu.VMEM((BATCH,), _i32),
    ] + [pltpu.SemaphoreType.DMA] * 3,
    compiler_params=_params,
)(_body0)

_extract1 = functools.partial(
    pl.kernel,
    out_type=_sdt((OUTW,), _f32),
    mesh=_mesh,
    scratch_types=[
        pltpu.VMEM((BATCH,), _i32),
        pltpu.VMEM((2, 2, MEM), _f32),
        pltpu.VMEM((BLKW,), _f32),
    ] + [pltpu.SemaphoreType.DMA] * 3,
    compiler_params=_params,
)(_body1)


def kernel(state0_buffer, actions_buffer, rewards_buffer, done_buffer,
           state1_buffer, priorities, indices, errors):
    bc_i = lambda x: lax.bitcast_convert_type(x, _i32)
    bc_f = lambda x: lax.bitcast_convert_type(x, _f32)
    xt0 = jnp.transpose(state0_buffer, (2, 3, 1, 0)).reshape(YX, C, MEM)
    xt1 = jnp.transpose(state1_buffer, (2, 3, 1, 0)).reshape(YX, C, MEM)
    idx = indices.astype(_i32)
    s0o, ao, ro, do_, wo, po = _extract0(
        xt0, bc_i(priorities), idx, bc_i(errors),
        actions_buffer.astype(_i32), bc_i(rewards_buffer),
        bc_i(done_buffer))
    s1o = _extract1(xt1, idx)

    def to_out(o):
        return (o.reshape(84, 84, 8, C, 128)
                 .transpose(2, 4, 3, 0, 1)
                 .reshape(BATCH, C, 84, 84))

    return (to_out(s0o), ao, bc_f(ro), bc_f(do_),
            to_out(s1o), bc_f(wo), bc_f(po))


# speedup vs baseline: 4.3482x; 2.0737x over previous
import functools

import jax
import jax.numpy as jnp
from jax import lax
from jax.experimental import pallas as pl
from jax.experimental.pallas import tpu as pltpu
from jax.experimental.pallas import tpu_sc as plsc

MEM = 10000
MEMP = 10112
BATCH = 1024
C = 4
YX = 84 * 84
BLK = 16
NBLK = YX // BLK
NC = 2
NS = 16
NW = NC * NS
MAXB = -(-NBLK // NW)
LANES = 16
NVEC = BATCH // LANES
SLABW = BATCH * C
BLKW = BLK * SLABW
OUTW = YX * SLABW
CORR = 0.1

_f32 = jnp.float32
_i32 = jnp.int32


_DBY = 16
_DROWS = _DBY * C * (MEMP // 128)


def _detile_body(xin, xout):
    v = xin[...]
    v = jnp.pad(v, ((0, 0), (0, 0), (0, MEMP - MEM)))
    xout[...] = v.reshape(_DROWS, 128)


def _detile(xt):
    return pl.pallas_call(
        _detile_body,
        grid=(YX // _DBY,),
        in_specs=[pl.BlockSpec((_DBY, C, MEM), lambda i: (i, 0, 0))],
        out_specs=pl.BlockSpec((_DROWS, 128), lambda i: (i, 0)),
        out_shape=jax.ShapeDtypeStruct((YX * C * (MEMP // 128), 128), _f32),
    )(xt)


def _rsqrt(x):
    xi = plsc.bitcast(x, _i32)
    yi = jnp.int32(0x5F3759DF) - lax.shift_right_arithmetic(xi, 1)
    y = plsc.bitcast(yi, _f32)
    for _ in range(3):
        y = y * (1.5 - 0.5 * x * y * y)
    return y


def _extract_blocks(xt, out, idx_v, pieces, accum, g0, g1, ws, tid):
    def issue(yx, half, slot, sem):
        return pltpu.async_copy(
            xt.at[yx, pl.ds(half * 2, 2), :], pieces.at[slot], sem)

    def gather_piece(slot, c0, roff):
        dc0 = jnp.zeros((LANES,), _i32)
        dc1 = jnp.ones((LANES,), _i32)

        def vbody(v, _):
            iv = idx_v[pl.ds(v * LANES, LANES)]
            base = roff + ((v >> 3) * 512 + (v & 7) * 16) + c0 * 128
            g = plsc.load_gather(pieces.at[slot], [dc0, iv])
            accum[pl.ds(base, LANES)] = g
            g2 = plsc.load_gather(pieces.at[slot], [dc1, iv])
            accum[pl.ds(base + 128, LANES)] = g2
            return 0

        lax.fori_loop(0, NVEC, vbody, 0, unroll=4)

    def block_body(j, _):
        b = tid + j * NW

        @pl.when(b < NBLK)
        def _():
            issue(b * BLK, 0, 0, g0)
            @pl.when(j > 0)
            def _():
                pltpu.make_async_copy(accum, out.at[pl.ds(0, BLKW)], ws).wait()

            def rbody(r, _):
                yx = b * BLK + r
                roff = r * 4096
                d1 = issue(yx, 1, 1, g1)
                pltpu.make_async_copy(xt.at[yx, pl.ds(0, 2), :],
                                      pieces.at[0], g0).wait()
                gather_piece(0, 0, roff)

                @pl.when(r < BLK - 1)
                def _():
                    issue(yx + 1, 0, 0, g0)

                d1.wait()
                gather_piece(1, 2, roff)
                return 0

            lax.fori_loop(0, BLK, rbody, 0)
            pltpu.async_copy(accum, out.at[pl.ds(b * BLKW, BLKW)], ws)

        return 0

    lax.fori_loop(0, MAXB, block_body, 0)
    pltpu.make_async_copy(accum, out.at[pl.ds(0, BLKW)], ws).wait()


def _body0(xt, pri, idxf, err, act, rew, don,
           out, o_a, o_r, o_d, o_w, o_p,
           idx_v, pieces, accum, table, vals, wbuf,
           g0, g1, ws):
    c = lax.axis_index("c")
    s = lax.axis_index("s")
    tid = s * NC + c
    lane = lax.iota(_i32, LANES)

    pltpu.sync_copy(idxf, idx_v)

    @pl.when(tid == 0)
    def _weights_and_scatter():
        pltpu.sync_copy(pri, table)
        pltpu.sync_copy(err, vals)
        m = jnp.zeros((LANES,), _f32)
        for v in range(NVEC):
            iv = idx_v[pl.ds(v * LANES, LANES)]
            p = plsc.bitcast(plsc.load_gather(table, [iv]), _f32)
            w = _rsqrt(jnp.float32(MEM) * p)
            wbuf[pl.ds(v * LANES, LANES)] = plsc.bitcast(w, _i32)
            m = jnp.maximum(m, w)
        mx = jnp.max(m) * jnp.ones((LANES,), _f32)
        for v in range(NVEC):
            w = plsc.bitcast(wbuf[pl.ds(v * LANES, LANES)], _f32)
            wbuf[pl.ds(v * LANES, LANES)] = plsc.bitcast(w / mx, _i32)
        pltpu.sync_copy(wbuf, o_w)
        for v in range(NVEC):
            iv = idx_v[pl.ds(v * LANES, LANES)]
            e = plsc.bitcast(vals[pl.ds(v * LANES, LANES)], _f32)
            nv = plsc.bitcast(jnp.abs(e) + CORR, _i32)
            for l in range(LANES):
                plsc.store_scatter(table, [iv], nv, mask=lane == l)
        pltpu.sync_copy(table, o_p)

    def _small_gather(src_hbm, dst_hbm):
        pltpu.sync_copy(src_hbm, table)
        for v in range(NVEC):
            iv = idx_v[pl.ds(v * LANES, LANES)]
            wbuf[pl.ds(v * LANES, LANES)] = plsc.load_gather(table, [iv])
        pltpu.sync_copy(wbuf, dst_hbm)

    @pl.when(tid == 1)
    def _actions():
        _small_gather(act, o_a)

    @pl.when(tid == 2)
    def _rewards():
        _small_gather(rew, o_r)

    @pl.when(tid == 3)
    def _done():
        _small_gather(don, o_d)

    _extract_blocks(xt, out, idx_v, pieces, accum, g0, g1, ws, tid)


def _body1(xt, idxf, out, idx_v, pieces, accum, g0, g1, ws):
    c = lax.axis_index("c")
    s = lax.axis_index("s")
    tid = s * NC + c
    pltpu.sync_copy(idxf, idx_v)
    _extract_blocks(xt, out, idx_v, pieces, accum, g0, g1, ws, tid)


_sdt = jax.ShapeDtypeStruct
_mesh = plsc.VectorSubcoreMesh(
    core_axis_name="c", subcore_axis_name="s",
    num_cores=NC, num_subcores=NS)
_params = pltpu.CompilerParams(
    needs_layout_passes=False, use_tc_tiling_on_sc=False)

_extract0 = functools.partial(
    pl.kernel,
    out_type=(
        _sdt((OUTW,), _f32),
        _sdt((BATCH,), _i32),
        _sdt((BATCH,), _i32),
        _sdt((BATCH,), _i32),
        _sdt((BATCH,), _i32),
        _sdt((MEM,), _i32),
    ),
    mesh=_mesh,
    scratch_types=[
        pltpu.VMEM((BATCH,), _i32),
        pltpu.VMEM((2, 2, MEMP), _f32),
        pltpu.VMEM((BLKW,), _f32),
        pltpu.VMEM((MEM,), _i32),
        pltpu.VMEM((BATCH,), _i32),
        pltpu.VMEM((BATCH,), _i32),
    ] + [pltpu.SemaphoreType.DMA] * 3,
    compiler_params=_params,
)(_body0)

_extract1 = functools.partial(
    pl.kernel,
    out_type=_sdt((OUTW,), _f32),
    mesh=_mesh,
    scratch_types=[
        pltpu.VMEM((BATCH,), _i32),
        pltpu.VMEM((2, 2, MEMP), _f32),
        pltpu.VMEM((BLKW,), _f32),
    ] + [pltpu.SemaphoreType.DMA] * 3,
    compiler_params=_params,
)(_body1)


def kernel(state0_buffer, actions_buffer, rewards_buffer, done_buffer,
           state1_buffer, priorities, indices, errors):
    bc_i = lambda x: lax.bitcast_convert_type(x, _i32)
    bc_f = lambda x: lax.bitcast_convert_type(x, _f32)
    xt0 = jnp.transpose(state0_buffer, (2, 3, 1, 0)).reshape(YX, C, MEM)
    xt1 = jnp.transpose(state1_buffer, (2, 3, 1, 0)).reshape(YX, C, MEM)
    lin0 = _detile(xt0).reshape(YX, C, MEMP)
    lin1 = _detile(xt1).reshape(YX, C, MEMP)
    idx = indices.astype(_i32)
    s0o, ao, ro, do_, wo, po = _extract0(
        lin0, bc_i(priorities), idx, bc_i(errors),
        actions_buffer.astype(_i32), bc_i(rewards_buffer),
        bc_i(done_buffer))
    s1o = _extract1(lin1, idx)

    def to_out(o):
        return (o.reshape(84, 84, 8, C, 128)
                 .transpose(2, 4, 3, 0, 1)
                 .reshape(BATCH, C, 84, 84))

    return (to_out(s0o), ao, bc_f(ro), bc_f(do_),
            to_out(s1o), bc_f(wo), bc_f(po))


# speedup vs baseline: 4.3496x; 1.0003x over previous
import functools

import jax
import jax.numpy as jnp
from jax import lax
from jax.experimental import pallas as pl
from jax.experimental.pallas import tpu as pltpu
from jax.experimental.pallas import tpu_sc as plsc

MEM = 10000
MEMP = 10112
BATCH = 1024
C = 4
YX = 84 * 84
BLK = 16
NBLK = YX // BLK
NC = 2
NS = 16
NW = NC * NS
MAXB = -(-NBLK // NW)
LANES = 16
NVEC = BATCH // LANES
SLABW = BATCH * C
BLKW = BLK * SLABW
OUTW = YX * SLABW
CORR = 0.1

_f32 = jnp.float32
_i32 = jnp.int32


_DBY = 16
_DROWS = _DBY * C * (MEMP // 128)


def _detile_body(xin, xout):
    v = xin[...]
    v = jnp.pad(v, ((0, 0), (0, 0), (0, MEMP - MEM)))
    xout[...] = v.reshape(_DROWS, 128)


def _detile(xt):
    return pl.pallas_call(
        _detile_body,
        grid=(YX // _DBY,),
        in_specs=[pl.BlockSpec((_DBY, C, MEM), lambda i: (i, 0, 0))],
        out_specs=pl.BlockSpec((_DROWS, 128), lambda i: (i, 0)),
        out_shape=jax.ShapeDtypeStruct((YX * C * (MEMP // 128), 128), _f32),
    )(xt)


def _rsqrt(x):
    xi = plsc.bitcast(x, _i32)
    yi = jnp.int32(0x5F3759DF) - lax.shift_right_arithmetic(xi, 1)
    y = plsc.bitcast(yi, _f32)
    for _ in range(3):
        y = y * (1.5 - 0.5 * x * y * y)
    return y


def _extract_blocks(xt, out, idx_v, pieces, accum, g0, g1, ws, tid):
    def issue(yx, half, slot, sem):
        return pltpu.async_copy(
            xt.at[yx, pl.ds(half * 2, 2), :], pieces.at[slot], sem)

    def gather_piece(slot, c0, roff):
        dc0 = jnp.zeros((LANES,), _i32)
        dc1 = jnp.ones((LANES,), _i32)

        def vbody(v, _):
            iv = idx_v[pl.ds(v * LANES, LANES)]
            base = roff + ((v >> 3) * 512 + (v & 7) * 16) + c0 * 128
            g = plsc.load_gather(pieces.at[slot], [dc0, iv])
            accum[pl.ds(base, LANES)] = g
            g2 = plsc.load_gather(pieces.at[slot], [dc1, iv])
            accum[pl.ds(base + 128, LANES)] = g2
            return 0

        lax.fori_loop(0, NVEC, vbody, 0, unroll=4)

    def block_body(j, _):
        b = tid + j * NW

        @pl.when(b < NBLK)
        def _():
            issue(b * BLK, 0, 0, g0)
            @pl.when(j > 0)
            def _():
                pltpu.make_async_copy(accum, out.at[pl.ds(0, BLKW)], ws).wait()

            def rbody(r, _):
                yx = b * BLK + r
                roff = r * 4096
                d1 = issue(yx, 1, 1, g1)
                pltpu.make_async_copy(xt.at[yx, pl.ds(0, 2), :],
                                      pieces.at[0], g0).wait()
                gather_piece(0, 0, roff)

                @pl.when(r < BLK - 1)
                def _():
                    issue(yx + 1, 0, 0, g0)

                d1.wait()
                gather_piece(1, 2, roff)
                return 0

            lax.fori_loop(0, BLK, rbody, 0)
            pltpu.async_copy(accum, out.at[pl.ds(b * BLKW, BLKW)], ws)

        return 0

    lax.fori_loop(0, MAXB, block_body, 0)
    pltpu.make_async_copy(accum, out.at[pl.ds(0, BLKW)], ws).wait()


def _body0(xt, pri, idxf, err, act, rew, don,
           out, o_a, o_r, o_d, o_w, o_p,
           idx_v, pieces, accum, table, vals, wbuf,
           g0, g1, ws):
    c = lax.axis_index("c")
    s = lax.axis_index("s")
    tid = s * NC + c
    lane = lax.iota(_i32, LANES)

    pltpu.sync_copy(idxf, idx_v)

    @pl.when(tid == 0)
    def _weights_and_scatter():
        pltpu.sync_copy(pri, table)
        pltpu.sync_copy(err, vals)

        def wbody(v, m):
            iv = idx_v[pl.ds(v * LANES, LANES)]
            p = plsc.bitcast(plsc.load_gather(table, [iv]), _f32)
            w = _rsqrt(jnp.float32(MEM) * p)
            wbuf[pl.ds(v * LANES, LANES)] = plsc.bitcast(w, _i32)
            return jnp.maximum(m, w)

        m = lax.fori_loop(0, NVEC, wbody, jnp.zeros((LANES,), _f32))
        mx = jnp.max(m) * jnp.ones((LANES,), _f32)

        def nbody(v, _):
            w = plsc.bitcast(wbuf[pl.ds(v * LANES, LANES)], _f32)
            wbuf[pl.ds(v * LANES, LANES)] = plsc.bitcast(w / mx, _i32)
            return 0

        lax.fori_loop(0, NVEC, nbody, 0)
        pltpu.sync_copy(wbuf, o_w)

        def sbody(v, _):
            iv = idx_v[pl.ds(v * LANES, LANES)]
            e = plsc.bitcast(vals[pl.ds(v * LANES, LANES)], _f32)
            nv = plsc.bitcast(jnp.abs(e) + CORR, _i32)
            for l in range(LANES):
                plsc.store_scatter(table, [iv], nv, mask=lane == l)
            return 0

        lax.fori_loop(0, NVEC, sbody, 0)
        pltpu.sync_copy(table, o_p)

    def _small_gather(src_hbm, dst_hbm):
        pltpu.sync_copy(src_hbm, table)

        def gbody(v, _):
            iv = idx_v[pl.ds(v * LANES, LANES)]
            wbuf[pl.ds(v * LANES, LANES)] = plsc.load_gather(table, [iv])
            return 0

        lax.fori_loop(0, NVEC, gbody, 0)
        pltpu.sync_copy(wbuf, dst_hbm)

    @pl.when(tid == 1)
    def _actions():
        _small_gather(act, o_a)

    @pl.when(tid == 2)
    def _rewards():
        _small_gather(rew, o_r)

    @pl.when(tid == 3)
    def _done():
        _small_gather(don, o_d)

    _extract_blocks(xt, out, idx_v, pieces, accum, g0, g1, ws, tid)


def _body1(xt, idxf, out, idx_v, pieces, accum, g0, g1, ws):
    c = lax.axis_index("c")
    s = lax.axis_index("s")
    tid = s * NC + c
    pltpu.sync_copy(idxf, idx_v)
    _extract_blocks(xt, out, idx_v, pieces, accum, g0, g1, ws, tid)


_sdt = jax.ShapeDtypeStruct
_mesh = plsc.VectorSubcoreMesh(
    core_axis_name="c", subcore_axis_name="s",
    num_cores=NC, num_subcores=NS)
_params = pltpu.CompilerParams(
    needs_layout_passes=False, use_tc_tiling_on_sc=False)

_extract0 = functools.partial(
    pl.kernel,
    out_type=(
        _sdt((OUTW,), _f32),
        _sdt((BATCH,), _i32),
        _sdt((BATCH,), _i32),
        _sdt((BATCH,), _i32),
        _sdt((BATCH,), _i32),
        _sdt((MEM,), _i32),
    ),
    mesh=_mesh,
    scratch_types=[
        pltpu.VMEM((BATCH,), _i32),
        pltpu.VMEM((2, 2, MEMP), _f32),
        pltpu.VMEM((BLKW,), _f32),
        pltpu.VMEM((MEM,), _i32),
        pltpu.VMEM((BATCH,), _i32),
        pltpu.VMEM((BATCH,), _i32),
    ] + [pltpu.SemaphoreType.DMA] * 3,
    compiler_params=_params,
)(_body0)

_extract1 = functools.partial(
    pl.kernel,
    out_type=_sdt((OUTW,), _f32),
    mesh=_mesh,
    scratch_types=[
        pltpu.VMEM((BATCH,), _i32),
        pltpu.VMEM((2, 2, MEMP), _f32),
        pltpu.VMEM((BLKW,), _f32),
    ] + [pltpu.SemaphoreType.DMA] * 3,
    compiler_params=_params,
)(_body1)


def kernel(state0_buffer, actions_buffer, rewards_buffer, done_buffer,
           state1_buffer, priorities, indices, errors):
    bc_i = lambda x: lax.bitcast_convert_type(x, _i32)
    bc_f = lambda x: lax.bitcast_convert_type(x, _f32)
    xt0 = jnp.transpose(state0_buffer, (2, 3, 1, 0)).reshape(YX, C, MEM)
    xt1 = jnp.transpose(state1_buffer, (2, 3, 1, 0)).reshape(YX, C, MEM)
    lin0 = _detile(xt0).reshape(YX, C, MEMP)
    lin1 = _detile(xt1).reshape(YX, C, MEMP)
    idx = indices.astype(_i32)
    s0o, ao, ro, do_, wo, po = _extract0(
        lin0, bc_i(priorities), idx, bc_i(errors),
        actions_buffer.astype(_i32), bc_i(rewards_buffer),
        bc_i(done_buffer))
    s1o = _extract1(lin1, idx)

    def to_out(o):
        return (o.reshape(84, 84, 8, C, 128)
                 .transpose(2, 4, 3, 0, 1)
                 .reshape(BATCH, C, 84, 84))

    return (to_out(s0o), ao, bc_f(ro), bc_f(do_),
            to_out(s1o), bc_f(wo), bc_f(po))


# speedup vs baseline: 4.4034x; 1.0124x over previous
import functools

import jax
import jax.numpy as jnp
from jax import lax
from jax.experimental import pallas as pl
from jax.experimental.pallas import tpu as pltpu
from jax.experimental.pallas import tpu_sc as plsc

MEM = 10000
MEMP = 10112
BATCH = 1024
C = 4
YX = 84 * 84
BLK = 16
NBLK = YX // BLK
NC = 2
NS = 16
NW = NC * NS
MAXB = -(-NBLK // NW)
LANES = 16
NVEC = BATCH // LANES
SLABW = BATCH * C
BLKW = BLK * SLABW
OUTW = YX * SLABW
CORR = 0.1

_f32 = jnp.float32
_i32 = jnp.int32


_DBY = 16
_DROWS = _DBY * C * (MEMP // 128)
NBLKA = 224
NBLKB = NBLK - NBLKA


def _detile_body(xin, xout):
    v = xin[...]
    v = jnp.pad(v, ((0, 0), (0, 0), (0, MEMP - MEM)))
    xout[...] = v.reshape(_DROWS, 128)


def _make_detile(nsteps, offset):
    def run(xt):
        return pl.pallas_call(
            _detile_body,
            grid=(nsteps,),
            in_specs=[pl.BlockSpec((_DBY, C, MEM),
                                   lambda i: (i + offset, 0, 0))],
            out_specs=pl.BlockSpec((_DROWS, 128), lambda i: (i, 0)),
            out_shape=jax.ShapeDtypeStruct((nsteps * _DROWS, 128), _f32),
        )(xt).reshape(nsteps * _DBY, C, MEMP)

    return run


_detile_a = _make_detile(NBLKA, 0)
_detile_b = _make_detile(NBLKB, NBLKA)


def _rsqrt(x):
    xi = plsc.bitcast(x, _i32)
    yi = jnp.int32(0x5F3759DF) - lax.shift_right_arithmetic(xi, 1)
    y = plsc.bitcast(yi, _f32)
    for _ in range(3):
        y = y * (1.5 - 0.5 * x * y * y)
    return y


def _extract_blocks(xt, out, idx_v, pieces, accum, g0, g1, ws, tid, nblk):
    def issue(yx, half, slot, sem):
        return pltpu.async_copy(
            xt.at[yx, pl.ds(half * 2, 2), :], pieces.at[slot], sem)

    def gather_piece(slot, c0, roff):
        dc0 = jnp.zeros((LANES,), _i32)
        dc1 = jnp.ones((LANES,), _i32)

        def vbody(v, _):
            iv = idx_v[pl.ds(v * LANES, LANES)]
            base = roff + ((v >> 3) * 512 + (v & 7) * 16) + c0 * 128
            g = plsc.load_gather(pieces.at[slot], [dc0, iv])
            accum[pl.ds(base, LANES)] = g
            g2 = plsc.load_gather(pieces.at[slot], [dc1, iv])
            accum[pl.ds(base + 128, LANES)] = g2
            return 0

        lax.fori_loop(0, NVEC, vbody, 0, unroll=4)

    def block_body(j, _):
        b = tid + j * NW

        @pl.when(b < nblk)
        def _():
            issue(b * BLK, 0, 0, g0)
            @pl.when(j > 0)
            def _():
                pltpu.make_async_copy(accum, out.at[pl.ds(0, BLKW)], ws).wait()

            def rbody(r, _):
                yx = b * BLK + r
                roff = r * 4096
                d1 = issue(yx, 1, 1, g1)
                pltpu.make_async_copy(xt.at[yx, pl.ds(0, 2), :],
                                      pieces.at[0], g0).wait()
                gather_piece(0, 0, roff)

                @pl.when(r < BLK - 1)
                def _():
                    issue(yx + 1, 0, 0, g0)

                d1.wait()
                gather_piece(1, 2, roff)
                return 0

            lax.fori_loop(0, BLK, rbody, 0)
            pltpu.async_copy(accum, out.at[pl.ds(b * BLKW, BLKW)], ws)

        return 0

    lax.fori_loop(0, -(-nblk // NW), block_body, 0)
    pltpu.make_async_copy(accum, out.at[pl.ds(0, BLKW)], ws).wait()


def _body0(xt, pri, idxf, err, act, rew, don,
           out, o_a, o_r, o_d, o_w, o_p,
           idx_v, pieces, accum, table, vals, wbuf,
           g0, g1, ws):
    c = lax.axis_index("c")
    s = lax.axis_index("s")
    tid = s * NC + c
    lane = lax.iota(_i32, LANES)

    pltpu.sync_copy(idxf, idx_v)

    @pl.when(tid == 0)
    def _weights_and_scatter():
        pltpu.sync_copy(pri, table)
        pltpu.sync_copy(err, vals)

        def wbody(v, m):
            iv = idx_v[pl.ds(v * LANES, LANES)]
            p = plsc.bitcast(plsc.load_gather(table, [iv]), _f32)
            w = _rsqrt(jnp.float32(MEM) * p)
            wbuf[pl.ds(v * LANES, LANES)] = plsc.bitcast(w, _i32)
            return jnp.maximum(m, w)

        m = lax.fori_loop(0, NVEC, wbody, jnp.zeros((LANES,), _f32))
        mx = jnp.max(m) * jnp.ones((LANES,), _f32)

        def nbody(v, _):
            w = plsc.bitcast(wbuf[pl.ds(v * LANES, LANES)], _f32)
            wbuf[pl.ds(v * LANES, LANES)] = plsc.bitcast(w / mx, _i32)
            return 0

        lax.fori_loop(0, NVEC, nbody, 0)
        pltpu.sync_copy(wbuf, o_w)

        def sbody(v, _):
            iv = idx_v[pl.ds(v * LANES, LANES)]
            e = plsc.bitcast(vals[pl.ds(v * LANES, LANES)], _f32)
            nv = plsc.bitcast(jnp.abs(e) + CORR, _i32)
            for l in range(LANES):
                plsc.store_scatter(table, [iv], nv, mask=lane == l)
            return 0

        lax.fori_loop(0, NVEC, sbody, 0)
        pltpu.sync_copy(table, o_p)

    def _small_gather(src_hbm, dst_hbm):
        pltpu.sync_copy(src_hbm, table)

        def gbody(v, _):
            iv = idx_v[pl.ds(v * LANES, LANES)]
            wbuf[pl.ds(v * LANES, LANES)] = plsc.load_gather(table, [iv])
            return 0

        lax.fori_loop(0, NVEC, gbody, 0)
        pltpu.sync_copy(wbuf, dst_hbm)

    @pl.when(tid == 1)
    def _actions():
        _small_gather(act, o_a)

    @pl.when(tid == 2)
    def _rewards():
        _small_gather(rew, o_r)

    @pl.when(tid == 3)
    def _done():
        _small_gather(don, o_d)

    _extract_blocks(xt, out, idx_v, pieces, accum, g0, g1, ws, tid, NBLKA)


def _make_body1(nblk):
    def _body1(xt, idxf, out, idx_v, pieces, accum, g0, g1, ws):
        c = lax.axis_index("c")
        s = lax.axis_index("s")
        tid = s * NC + c
        pltpu.sync_copy(idxf, idx_v)
        _extract_blocks(xt, out, idx_v, pieces, accum, g0, g1, ws, tid, nblk)

    return _body1


_sdt = jax.ShapeDtypeStruct
_mesh = plsc.VectorSubcoreMesh(
    core_axis_name="c", subcore_axis_name="s",
    num_cores=NC, num_subcores=NS)
_params = pltpu.CompilerParams(
    needs_layout_passes=False, use_tc_tiling_on_sc=False)

_extract0 = functools.partial(
    pl.kernel,
    out_type=(
        _sdt((NBLKA * BLKW,), _f32),
        _sdt((BATCH,), _i32),
        _sdt((BATCH,), _i32),
        _sdt((BATCH,), _i32),
        _sdt((BATCH,), _i32),
        _sdt((MEM,), _i32),
    ),
    mesh=_mesh,
    scratch_types=[
        pltpu.VMEM((BATCH,), _i32),
        pltpu.VMEM((2, 2, MEMP), _f32),
        pltpu.VMEM((BLKW,), _f32),
        pltpu.VMEM((MEM,), _i32),
        pltpu.VMEM((BATCH,), _i32),
        pltpu.VMEM((BATCH,), _i32),
    ] + [pltpu.SemaphoreType.DMA] * 3,
    compiler_params=_params,
)(_body0)

def _make_extract_plain(nblk):
    return functools.partial(
        pl.kernel,
        out_type=_sdt((nblk * BLKW,), _f32),
        mesh=_mesh,
        scratch_types=[
            pltpu.VMEM((BATCH,), _i32),
            pltpu.VMEM((2, 2, MEMP), _f32),
            pltpu.VMEM((BLKW,), _f32),
        ] + [pltpu.SemaphoreType.DMA] * 3,
        compiler_params=_params,
    )(_make_body1(nblk))


_extract_a = _make_extract_plain(NBLKA)
_extract_b = _make_extract_plain(NBLKB)


def kernel(state0_buffer, actions_buffer, rewards_buffer, done_buffer,
           state1_buffer, priorities, indices, errors):
    bc_i = lambda x: lax.bitcast_convert_type(x, _i32)
    bc_f = lambda x: lax.bitcast_convert_type(x, _f32)
    xt0 = jnp.transpose(state0_buffer, (2, 3, 1, 0)).reshape(YX, C, MEM)
    xt1 = jnp.transpose(state1_buffer, (2, 3, 1, 0)).reshape(YX, C, MEM)
    idx = indices.astype(_i32)
    lin0a = _detile_a(xt0)
    s0a, ao, ro, do_, wo, po = _extract0(
        lin0a, bc_i(priorities), idx, bc_i(errors),
        actions_buffer.astype(_i32), bc_i(rewards_buffer),
        bc_i(done_buffer))
    lin0b = _detile_b(xt0)
    s0b = _extract_b(lin0b, idx)
    lin1a = _detile_a(xt1)
    s1a = _extract_a(lin1a, idx)
    lin1b = _detile_b(xt1)
    s1b = _extract_b(lin1b, idx)
    s0o = jnp.concatenate([s0a, s0b])
    s1o = jnp.concatenate([s1a, s1b])

    def to_out(o):
        return (o.reshape(84, 84, 8, C, 128)
                 .transpose(2, 4, 3, 0, 1)
                 .reshape(BATCH, C, 84, 84))

    return (to_out(s0o), ao, bc_f(ro), bc_f(do_),
            to_out(s1o), bc_f(wo), bc_f(po))


# speedup vs baseline: 4.5879x; 1.0419x over previous
import functools

import jax
import jax.numpy as jnp
from jax import lax
from jax.experimental import pallas as pl
from jax.experimental.pallas import tpu as pltpu
from jax.experimental.pallas import tpu_sc as plsc

MEM = 10000
MEMP = 10112
BATCH = 1024
C = 4
YX = 84 * 84
BLK = 16
NBLK = YX // BLK
NC = 2
NS = 16
NW = NC * NS
MAXB = -(-NBLK // NW)
LANES = 16
NVEC = BATCH // LANES
SLABW = BATCH * C
BLKW = BLK * SLABW
OUTW = YX * SLABW
CORR = 0.1

_f32 = jnp.float32
_i32 = jnp.int32


_DBY = 48
_DROWS = _DBY * C * (MEMP // 128)
NBLKA = 210
NBLKB = NBLK - NBLKA


def _detile_body(xin, xout):
    v = xin[...]
    v = jnp.pad(v, ((0, 0), (0, 0), (0, MEMP - MEM)))
    xout[...] = v.reshape(_DROWS, 128)


def _make_detile(nsteps, offset):
    def run(xt):
        return pl.pallas_call(
            _detile_body,
            grid=(nsteps,),
            in_specs=[pl.BlockSpec((_DBY, C, MEM),
                                   lambda i: (i + offset, 0, 0))],
            out_specs=pl.BlockSpec((_DROWS, 128), lambda i: (i, 0)),
            out_shape=jax.ShapeDtypeStruct((nsteps * _DROWS, 128), _f32),
        )(xt).reshape(nsteps * _DBY, C, MEMP)

    return run


_detile_a = _make_detile(NBLKA * BLK // _DBY, 0)
_detile_b = _make_detile(NBLKB * BLK // _DBY, NBLKA * BLK // _DBY)


def _rsqrt(x):
    xi = plsc.bitcast(x, _i32)
    yi = jnp.int32(0x5F3759DF) - lax.shift_right_arithmetic(xi, 1)
    y = plsc.bitcast(yi, _f32)
    for _ in range(3):
        y = y * (1.5 - 0.5 * x * y * y)
    return y


def _extract_blocks(xt, out, idx_v, pieces, accum, g0, g1, ws, tid, nblk):
    def issue(yx, half, slot, sem):
        return pltpu.async_copy(
            xt.at[yx, pl.ds(half * 2, 2), :], pieces.at[slot], sem)

    def gather_piece(slot, c0, roff):
        dc0 = jnp.zeros((LANES,), _i32)
        dc1 = jnp.ones((LANES,), _i32)

        def vbody(v, _):
            iv = idx_v[pl.ds(v * LANES, LANES)]
            base = roff + ((v >> 3) * 512 + (v & 7) * 16) + c0 * 128
            g = plsc.load_gather(pieces.at[slot], [dc0, iv])
            accum[pl.ds(base, LANES)] = g
            g2 = plsc.load_gather(pieces.at[slot], [dc1, iv])
            accum[pl.ds(base + 128, LANES)] = g2
            return 0

        lax.fori_loop(0, NVEC, vbody, 0, unroll=4)

    def block_body(j, _):
        b = tid + j * NW

        @pl.when(b < nblk)
        def _():
            issue(b * BLK, 0, 0, g0)
            @pl.when(j > 0)
            def _():
                pltpu.make_async_copy(accum, out.at[pl.ds(0, BLKW)], ws).wait()

            def rbody(r, _):
                yx = b * BLK + r
                roff = r * 4096
                d1 = issue(yx, 1, 1, g1)
                pltpu.make_async_copy(xt.at[yx, pl.ds(0, 2), :],
                                      pieces.at[0], g0).wait()
                gather_piece(0, 0, roff)

                @pl.when(r < BLK - 1)
                def _():
                    issue(yx + 1, 0, 0, g0)

                d1.wait()
                gather_piece(1, 2, roff)
                return 0

            lax.fori_loop(0, BLK, rbody, 0)
            pltpu.async_copy(accum, out.at[pl.ds(b * BLKW, BLKW)], ws)

        return 0

    lax.fori_loop(0, -(-nblk // NW), block_body, 0)
    pltpu.make_async_copy(accum, out.at[pl.ds(0, BLKW)], ws).wait()


def _body0(xt, pri, idxf, err, act, rew, don,
           out, o_a, o_r, o_d, o_w, o_p,
           idx_v, pieces, accum, table, vals, wbuf,
           g0, g1, ws):
    c = lax.axis_index("c")
    s = lax.axis_index("s")
    tid = s * NC + c
    lane = lax.iota(_i32, LANES)

    pltpu.sync_copy(idxf, idx_v)

    @pl.when(tid == 0)
    def _weights_and_scatter():
        pltpu.sync_copy(pri, table)
        pltpu.sync_copy(err, vals)

        def wbody(v, m):
            iv = idx_v[pl.ds(v * LANES, LANES)]
            p = plsc.bitcast(plsc.load_gather(table, [iv]), _f32)
            w = _rsqrt(jnp.float32(MEM) * p)
            wbuf[pl.ds(v * LANES, LANES)] = plsc.bitcast(w, _i32)
            return jnp.maximum(m, w)

        m = lax.fori_loop(0, NVEC, wbody, jnp.zeros((LANES,), _f32))
        mx = jnp.max(m) * jnp.ones((LANES,), _f32)

        def nbody(v, _):
            w = plsc.bitcast(wbuf[pl.ds(v * LANES, LANES)], _f32)
            wbuf[pl.ds(v * LANES, LANES)] = plsc.bitcast(w / mx, _i32)
            return 0

        lax.fori_loop(0, NVEC, nbody, 0)
        pltpu.sync_copy(wbuf, o_w)

        def sbody(v, _):
            iv = idx_v[pl.ds(v * LANES, LANES)]
            e = plsc.bitcast(vals[pl.ds(v * LANES, LANES)], _f32)
            nv = plsc.bitcast(jnp.abs(e) + CORR, _i32)
            for l in range(LANES):
                plsc.store_scatter(table, [iv], nv, mask=lane == l)
            return 0

        lax.fori_loop(0, NVEC, sbody, 0)
        pltpu.sync_copy(table, o_p)

    def _small_gather(src_hbm, dst_hbm):
        pltpu.sync_copy(src_hbm, table)

        def gbody(v, _):
            iv = idx_v[pl.ds(v * LANES, LANES)]
            wbuf[pl.ds(v * LANES, LANES)] = plsc.load_gather(table, [iv])
            return 0

        lax.fori_loop(0, NVEC, gbody, 0)
        pltpu.sync_copy(wbuf, dst_hbm)

    @pl.when(tid == 1)
    def _actions():
        _small_gather(act, o_a)

    @pl.when(tid == 2)
    def _rewards():
        _small_gather(rew, o_r)

    @pl.when(tid == 3)
    def _done():
        _small_gather(don, o_d)

    _extract_blocks(xt, out, idx_v, pieces, accum, g0, g1, ws, tid, NBLKA)


def _make_body1(nblk):
    def _body1(xt, idxf, out, idx_v, pieces, accum, g0, g1, ws):
        c = lax.axis_index("c")
        s = lax.axis_index("s")
        tid = s * NC + c
        pltpu.sync_copy(idxf, idx_v)
        _extract_blocks(xt, out, idx_v, pieces, accum, g0, g1, ws, tid, nblk)

    return _body1


_sdt = jax.ShapeDtypeStruct
_mesh = plsc.VectorSubcoreMesh(
    core_axis_name="c", subcore_axis_name="s",
    num_cores=NC, num_subcores=NS)
_params = pltpu.CompilerParams(
    needs_layout_passes=False, use_tc_tiling_on_sc=False)

_extract0 = functools.partial(
    pl.kernel,
    out_type=(
        _sdt((NBLKA * BLKW,), _f32),
        _sdt((BATCH,), _i32),
        _sdt((BATCH,), _i32),
        _sdt((BATCH,), _i32),
        _sdt((BATCH,), _i32),
        _sdt((MEM,), _i32),
    ),
    mesh=_mesh,
    scratch_types=[
        pltpu.VMEM((BATCH,), _i32),
        pltpu.VMEM((2, 2, MEMP), _f32),
        pltpu.VMEM((BLKW,), _f32),
        pltpu.VMEM((MEM,), _i32),
        pltpu.VMEM((BATCH,), _i32),
        pltpu.VMEM((BATCH,), _i32),
    ] + [pltpu.SemaphoreType.DMA] * 3,
    compiler_params=_params,
)(_body0)

def _make_extract_plain(nblk):
    return functools.partial(
        pl.kernel,
        out_type=_sdt((nblk * BLKW,), _f32),
        mesh=_mesh,
        scratch_types=[
            pltpu.VMEM((BATCH,), _i32),
            pltpu.VMEM((2, 2, MEMP), _f32),
            pltpu.VMEM((BLKW,), _f32),
        ] + [pltpu.SemaphoreType.DMA] * 3,
        compiler_params=_params,
    )(_make_body1(nblk))


_extract_a = _make_extract_plain(NBLKA)
_extract_b = _make_extract_plain(NBLKB)


def kernel(state0_buffer, actions_buffer, rewards_buffer, done_buffer,
           state1_buffer, priorities, indices, errors):
    bc_i = lambda x: lax.bitcast_convert_type(x, _i32)
    bc_f = lambda x: lax.bitcast_convert_type(x, _f32)
    xt0 = jnp.transpose(state0_buffer, (2, 3, 1, 0)).reshape(YX, C, MEM)
    xt1 = jnp.transpose(state1_buffer, (2, 3, 1, 0)).reshape(YX, C, MEM)
    idx = indices.astype(_i32)
    lin0a = _detile_a(xt0)
    s0a, ao, ro, do_, wo, po = _extract0(
        lin0a, bc_i(priorities), idx, bc_i(errors),
        actions_buffer.astype(_i32), bc_i(rewards_buffer),
        bc_i(done_buffer))
    lin0b = _detile_b(xt0)
    s0b = _extract_b(lin0b, idx)
    lin1a = _detile_a(xt1)
    s1a = _extract_a(lin1a, idx)
    lin1b = _detile_b(xt1)
    s1b = _extract_b(lin1b, idx)
    s0o = jnp.concatenate([s0a, s0b])
    s1o = jnp.concatenate([s1a, s1b])

    def to_out(o):
        return (o.reshape(84, 84, 8, C, 128)
                 .transpose(2, 4, 3, 0, 1)
                 .reshape(BATCH, C, 84, 84))

    return (to_out(s0o), ao, bc_f(ro), bc_f(do_),
            to_out(s1o), bc_f(wo), bc_f(po))
